# Initial kernel scaffold; baseline (speedup 1.0000x reference)
import jax
import jax.numpy as jnp
from jax.experimental import pallas as pl


def kernel(query, keyval, params):
    raise NotImplementedError("write your pallas kernel here")


# speedup vs baseline: 7.9816x; 7.9816x over previous
import functools

import jax
import jax.numpy as jnp
from jax.experimental import pallas as pl
from jax.experimental.pallas import tpu as pltpu

D = 1024
H = 4
DK = 64
DV = 64
M = 4
KC = 4
I = 2048
EPS = 1e-6
C = 64
MH = M * H
PW = 1152
F3 = 3 * H * DK


def _f32dot(a, b):
    return jnp.dot(a, b, preferred_element_type=jnp.float32)


def _dg(a, b, dims):
    return jax.lax.dot_general(a, b, (dims, ((), ())),
                               preferred_element_type=jnp.float32)


def _prologue_body(x_ref, nw_ref, w_ref, o_ref):
    x = x_ref[...]
    xn = x * jax.lax.rsqrt(jnp.mean(x * x, axis=-1, keepdims=True) + EPS)
    xn = xn * nw_ref[...]
    o_ref[...] = _f32dot(xn, w_ref[...])


def _prologue(x, norm_w, wcat):
    T = x.shape[0]
    bm = 256
    return pl.pallas_call(
        _prologue_body,
        grid=(T // bm,),
        in_specs=[
            pl.BlockSpec((bm, D), lambda i: (i, 0)),
            pl.BlockSpec((1, D), lambda i: (0, 0)),
            pl.BlockSpec((D, PW), lambda i: (0, 0)),
        ],
        out_specs=pl.BlockSpec((bm, PW), lambda i: (i, 0)),
        out_shape=jax.ShapeDtypeStruct((T, PW), jnp.float32),
    )(x, norm_w.reshape(1, D), wcat)


def _scan_body(p_ref, wcv_ref, o_ref, s_ref, hist_ref):
    c = pl.program_id(0)

    @pl.when(c == 0)
    def _():
        s_ref[...] = jnp.zeros_like(s_ref)
        hist_ref[...] = jnp.zeros_like(hist_ref)

    P = p_ref[...]
    raw = P[:, 0:F3]
    wcv = wcv_ref[...]

    xp = jnp.concatenate([hist_ref[...], raw], axis=0)
    y = (xp[0:C] * wcv[0:1] + xp[1:C + 1] * wcv[1:2]
         + xp[2:C + 2] * wcv[2:3] + xp[3:C + 3] * wcv[3:4])
    y = y * jax.nn.sigmoid(y)
    hist_ref[...] = raw[C - 3:C, :]

    qr = y[:, 0:256]
    kr = y[:, 256:512]
    v = y[:, 512:768]

    blk = ((jax.lax.broadcasted_iota(jnp.int32, (256, H), 0) // DK)
           == jax.lax.broadcasted_iota(jnp.int32, (256, H), 1)
           ).astype(jnp.float32)
    qs = jax.lax.rsqrt(_f32dot(qr * qr, blk) + 1e-6)
    ks = jax.lax.rsqrt(_f32dot(kr * kr, blk) + 1e-6)
    q = qr * _dg(qs, blk, ((1,), (1,)))
    k = kr * _dg(ks, blk, ((1,), (1,)))

    beta = jax.nn.sigmoid(P[:, 1024:1028])
    g = -jax.nn.softplus(P[:, 1028:1032])
    rl = P[:, 1032:1036]

    m_iota = jax.lax.broadcasted_iota(jnp.int32, (C, M), 1)
    mx1 = jnp.max(rl, axis=1, keepdims=True)
    idx1 = jnp.min(jnp.where(rl == mx1, m_iota, M), axis=1, keepdims=True)
    one1 = (m_iota == idx1)
    rl2 = jnp.where(one1, -1e30, rl)
    mx2 = jnp.max(rl2, axis=1, keepdims=True)
    idx2 = jnp.min(jnp.where(rl2 == mx2, m_iota, M), axis=1, keepdims=True)
    one2 = (m_iota == idx2)
    e2 = jnp.exp(mx2 - mx1)
    z = 1.0 + e2
    wfull = (one1.astype(jnp.float32) + one2.astype(jnp.float32) * e2) / z
    ind = (one1 | one2).astype(jnp.float32)

    rep = ((jax.lax.broadcasted_iota(jnp.int32, (H, MH), 1) % H)
           == jax.lax.broadcasted_iota(jnp.int32, (H, MH), 0)
           ).astype(jnp.float32)
    mem = ((jax.lax.broadcasted_iota(jnp.int32, (M, MH), 1) // H)
           == jax.lax.broadcasted_iota(jnp.int32, (M, MH), 0)
           ).astype(jnp.float32)
    b16 = _f32dot(beta, rep) * _f32dot(wfull, mem)
    la16 = _f32dot(g, rep) * _f32dot(ind, mem)

    ii = jax.lax.broadcasted_iota(jnp.int32, (C, C), 0)
    jj = jax.lax.broadcasted_iota(jnp.int32, (C, C), 1)
    tril = (ii >= jj).astype(jnp.float32)
    L16 = _f32dot(tril, la16)
    LT = _dg(la16, tril, ((0,), (1,)))

    ohs = []
    for h in range(H):
        Kh = k[:, h * DK:(h + 1) * DK]
        Qh = q[:, h * DK:(h + 1) * DK]
        Vh = v[:, h * DV:(h + 1) * DV]
        KK = _dg(Kh, Kh, ((1,), (1,)))
        QK = _dg(Qh, Kh, ((1,), (1,)))
        oh = jnp.zeros((C, DV), jnp.float32)
        for m in range(M):
            mh = m * H + h
            Lc = L16[:, mh:mh + 1]
            Lr = LT[mh:mh + 1, :]
            bc = b16[:, mh:mh + 1]
            wm = wfull[:, m:m + 1]
            Dm = jnp.exp(jnp.where(ii >= jj, Lc - Lr, -1e30))
            A = jnp.where(ii > jj, bc * Dm * KK, 0.0)
            N = -A
            S0 = s_ref[mh]
            Kbar = Kh * (bc * jnp.exp(Lc))
            X = bc * Vh - _f32dot(Kbar, S0)
            N2 = _f32dot(N, N)
            N4 = _f32dot(N2, N2)
            N8 = _f32dot(N4, N4)
            N16 = _f32dot(N8, N8)
            N32 = _f32dot(N16, N16)
            U = X + _f32dot(N32, X)
            U = U + _f32dot(N16, U)
            U = U + _f32dot(N8, U)
            U = U + _f32dot(N4, U)
            U = U + _f32dot(N2, U)
            U = U + _f32dot(N, U)
            oh = oh + wm * (jnp.exp(Lc) * _f32dot(Qh, S0)
                            + _f32dot(Dm * QK, U))
            LCs = L16[C - 1:C, mh:mh + 1]
            Kdec = Kh * jnp.exp(LCs - Lc)
            s_ref[mh] = jnp.exp(LCs) * S0 + _dg(Kdec, U, ((0,), (0,)))
        ohs.append(oh)

    o = jnp.concatenate(ohs, axis=1)
    ms = _f32dot(o * o, blk) * (1.0 / DV)
    o = o * _dg(jax.lax.rsqrt(ms + EPS), blk, ((1,), (1,)))
    gate = P[:, 768:1024]
    o_ref[...] = o * (gate * jax.nn.sigmoid(gate))


def _mom_scan(P, wcv):
    T = P.shape[0]
    return pl.pallas_call(
        _scan_body,
        grid=(T // C,),
        in_specs=[
            pl.BlockSpec((C, PW), lambda c: (c, 0)),
            pl.BlockSpec((KC, F3), lambda c: (0, 0)),
        ],
        out_specs=pl.BlockSpec((C, H * DV), lambda c: (c, 0)),
        out_shape=jax.ShapeDtypeStruct((T, H * DV), jnp.float32),
        scratch_shapes=[
            pltpu.VMEM((MH, DK, DV), jnp.float32),
            pltpu.VMEM((3, F3), jnp.float32),
        ],
    )(P, wcv)


def _mlp_body(og_ref, wo_ref, res_ref, nw_ref, wg_ref, wu_ref, wd_ref,
              out_ref, xn_ref):
    i = pl.program_id(1)

    @pl.when(i == 0)
    def _():
        r = _f32dot(og_ref[...], wo_ref[...]) + res_ref[...]
        xn = r * jax.lax.rsqrt(jnp.mean(r * r, axis=-1, keepdims=True) + EPS)
        xn_ref[...] = xn * nw_ref[...]
        out_ref[...] = r

    xn = xn_ref[...]
    a = _f32dot(xn, wg_ref[...])
    b = _f32dot(xn, wu_ref[...])
    out_ref[...] += _f32dot(a * jax.nn.sigmoid(a) * b, wd_ref[...])


def _mlp(og, wo_s, res, norm_w, mp):
    T = og.shape[0]
    bm = 256
    bi = 512
    return pl.pallas_call(
        _mlp_body,
        grid=(T // bm, I // bi),
        in_specs=[
            pl.BlockSpec((bm, H * DV), lambda t, i: (t, 0)),
            pl.BlockSpec((H * DV, D), lambda t, i: (0, 0)),
            pl.BlockSpec((bm, D), lambda t, i: (t, 0)),
            pl.BlockSpec((1, D), lambda t, i: (0, 0)),
            pl.BlockSpec((D, bi), lambda t, i: (0, i)),
            pl.BlockSpec((D, bi), lambda t, i: (0, i)),
            pl.BlockSpec((bi, D), lambda t, i: (i, 0)),
        ],
        out_specs=pl.BlockSpec((bm, D), lambda t, i: (t, 0)),
        out_shape=jax.ShapeDtypeStruct((T, D), jnp.float32),
        scratch_shapes=[pltpu.VMEM((bm, D), jnp.float32)],
    )(og, wo_s, res, norm_w.reshape(1, D), mp['wg'], mp['wu'], mp['wd'])


def _wcat(ap):
    pad = jnp.zeros((D, PW - (4 * 256 + 3 * M)), jnp.float32)
    return jnp.concatenate(
        [ap['wq'], ap['wk'], ap['wv'], ap['w_gate'],
         ap['w_beta'], ap['w_g'], ap['w_router'], pad], axis=1)


def _wcv(ap):
    return jnp.concatenate(
        [ap['q_conv'], ap['k_conv'], ap['v_conv']], axis=0).T


def _wo_scaled(ap):
    return ap['wo'] * jnp.tile(ap['o_norm_w'], H)[:, None]


def kernel(query, keyval, params):
    p = params
    x0 = query[0]
    kv = keyval[0]
    a1, a2 = p['qa_attn'], p['qkv_attn']

    P1 = _prologue(x0, p['qa_attn_norm'], _wcat(a1))
    OG1 = _mom_scan(P1, _wcv(a1))
    out1 = _mlp(OG1, _wo_scaled(a1), x0, p['qa_mlp_norm'], p['qa_mlp'])

    qkv = jnp.concatenate([kv, out1], axis=0)
    P2 = _prologue(qkv, p['qkv_norm'], _wcat(a2))
    OG2 = _mom_scan(P2, _wcv(a2))
    out2 = _mlp(OG2[kv.shape[0]:], _wo_scaled(a2), out1, p['ffn_norm'],
                p['ffn'])

    q_rl = P1[:, 1032:1032 + M][None]
    kv_rl = P2[:, 1032:1032 + M][None]
    return out2[None], q_rl, kv_rl


# speedup vs baseline: 11.2853x; 1.4139x over previous
import functools

import jax
import jax.numpy as jnp
from jax.experimental import pallas as pl
from jax.experimental.pallas import tpu as pltpu

D = 1024
H = 4
DK = 64
DV = 64
M = 4
KC = 4
I = 2048
EPS = 1e-6
C = 64
MH = M * H
PW = 1152
F3 = 3 * H * DK


def _f32dot(a, b):
    return jnp.dot(a, b, preferred_element_type=jnp.float32)


def _dg(a, b, dims):
    return jax.lax.dot_general(a, b, (dims, ((), ())),
                               preferred_element_type=jnp.float32)


def _prologue_body(x_ref, nw_ref, w_ref, o_ref):
    x = x_ref[...]
    xn = x * jax.lax.rsqrt(jnp.mean(x * x, axis=-1, keepdims=True) + EPS)
    xn = xn * nw_ref[...]
    o_ref[...] = _f32dot(xn, w_ref[...])


def _prologue(x, norm_w, wcat):
    T = x.shape[0]
    bm = 256
    return pl.pallas_call(
        _prologue_body,
        grid=(T // bm,),
        in_specs=[
            pl.BlockSpec((bm, D), lambda i: (i, 0)),
            pl.BlockSpec((1, D), lambda i: (0, 0)),
            pl.BlockSpec((D, PW), lambda i: (0, 0)),
        ],
        out_specs=pl.BlockSpec((bm, PW), lambda i: (i, 0)),
        out_shape=jax.ShapeDtypeStruct((T, PW), jnp.float32),
    )(x, norm_w.reshape(1, D), wcat)


NSUB = 2
S2 = NSUB * C


def _scan_body(p_ref, wcv_ref, o_ref, s_ref, hist_ref):
    c = pl.program_id(0)

    @pl.when(c == 0)
    def _():
        s_ref[...] = jnp.zeros_like(s_ref)
        hist_ref[...] = jnp.zeros_like(hist_ref)

    P = p_ref[...]
    raw = P[:, 0:F3]
    wcv = wcv_ref[...]

    xp = jnp.concatenate([hist_ref[...], raw], axis=0)
    y = (xp[0:S2] * wcv[0:1] + xp[1:S2 + 1] * wcv[1:2]
         + xp[2:S2 + 2] * wcv[2:3] + xp[3:S2 + 3] * wcv[3:4])
    y = y * jax.nn.sigmoid(y)
    hist_ref[...] = raw[S2 - 3:S2, :]

    qr = y[:, 0:256]
    kr = y[:, 256:512]
    v = y[:, 512:768]

    blk = ((jax.lax.broadcasted_iota(jnp.int32, (256, H), 0) // DK)
           == jax.lax.broadcasted_iota(jnp.int32, (256, H), 1)
           ).astype(jnp.float32)
    blkT = ((jax.lax.broadcasted_iota(jnp.int32, (H, 256), 1) // DK)
            == jax.lax.broadcasted_iota(jnp.int32, (H, 256), 0)
            ).astype(jnp.float32)
    qs = jax.lax.rsqrt(_f32dot(qr * qr, blk) + 1e-6)
    ks = jax.lax.rsqrt(_f32dot(kr * kr, blk) + 1e-6)
    q = qr * _f32dot(qs, blkT)
    k = kr * _f32dot(ks, blkT)

    beta = jax.nn.sigmoid(P[:, 1024:1028])
    g = -jax.nn.softplus(P[:, 1028:1032])
    rl = P[:, 1032:1036]

    m_iota = jax.lax.broadcasted_iota(jnp.int32, (S2, M), 1)
    mx1 = jnp.max(rl, axis=1, keepdims=True)
    idx1 = jnp.min(jnp.where(rl == mx1, m_iota, M), axis=1, keepdims=True)
    one1 = (m_iota == idx1)
    rl2 = jnp.where(one1, -1e30, rl)
    mx2 = jnp.max(rl2, axis=1, keepdims=True)
    idx2 = jnp.min(jnp.where(rl2 == mx2, m_iota, M), axis=1, keepdims=True)
    one2 = (m_iota == idx2)
    e2 = jnp.exp(mx2 - mx1)
    z = 1.0 + e2
    wfull = (one1.astype(jnp.float32) + one2.astype(jnp.float32) * e2) / z
    ind = (one1 | one2).astype(jnp.float32)

    rep = ((jax.lax.broadcasted_iota(jnp.int32, (H, MH), 1) % H)
           == jax.lax.broadcasted_iota(jnp.int32, (H, MH), 0)
           ).astype(jnp.float32)
    mem = ((jax.lax.broadcasted_iota(jnp.int32, (M, MH), 1) // H)
           == jax.lax.broadcasted_iota(jnp.int32, (M, MH), 0)
           ).astype(jnp.float32)
    b16 = _f32dot(beta, rep) * _f32dot(wfull, mem)
    la16 = _f32dot(g, rep) * _f32dot(ind, mem)

    ii2 = jax.lax.broadcasted_iota(jnp.int32, (S2, S2), 0)
    jj2 = jax.lax.broadcasted_iota(jnp.int32, (S2, S2), 1)
    same = (ii2 // C) == (jj2 // C)
    trilb = ((ii2 >= jj2) & same).astype(jnp.float32)
    triub = ((ii2 <= jj2) & same).astype(jnp.float32)
    L16 = _f32dot(trilb, la16)
    LT = _dg(la16, triub, ((0,), (0,)))

    ii = jax.lax.broadcasted_iota(jnp.int32, (C, C), 0)
    jj = jax.lax.broadcasted_iota(jnp.int32, (C, C), 1)
    low = ii >= jj
    lows = ii > jj

    out = []
    for s in range(NSUB):
        r0 = s * C
        ohs = []
        for h in range(H):
            Kh = k[r0:r0 + C, h * DK:(h + 1) * DK]
            Qh = q[r0:r0 + C, h * DK:(h + 1) * DK]
            Vh = v[r0:r0 + C, h * DV:(h + 1) * DV]
            KK = _dg(Kh, Kh, ((1,), (1,)))
            QK = _dg(Qh, Kh, ((1,), (1,)))
            S0 = s_ref[h]
            KS = _f32dot(Kh, S0)
            QS = _f32dot(Qh, S0)
            oh = jnp.zeros((C, DV), jnp.float32)
            Snew = []
            for m in range(M):
                mh = m * H + h
                Lc = L16[r0:r0 + C, mh:mh + 1]
                Lr = LT[mh:mh + 1, r0:r0 + C]
                bc = b16[r0:r0 + C, mh:mh + 1]
                wm = wfull[r0:r0 + C, m:m + 1]
                eLc = jnp.exp(Lc)
                Dm = jnp.exp(jnp.where(low, Lc - Lr, -1e30))
                N = jnp.where(lows, (-bc) * Dm * KK, 0.0)
                X = bc * Vh - (bc * eLc) * KS[:, m * DV:(m + 1) * DV]
                U = X + _f32dot(N, X)
                N2 = _f32dot(N, N)
                U = U + _f32dot(N2, U)
                N4 = _f32dot(N2, N2)
                U = U + _f32dot(N4, U)
                N8 = _f32dot(N4, N4)
                U = U + _f32dot(N8, U)
                N16 = _f32dot(N8, N8)
                U = U + _f32dot(N16, U)
                N32 = _f32dot(N16, N16)
                U = U + _f32dot(N32, U)
                oh = oh + wm * (eLc * QS[:, m * DV:(m + 1) * DV]
                                + _f32dot(Dm * QK, U))
                LCs = L16[r0 + C - 1:r0 + C, mh:mh + 1]
                Kdec = Kh * jnp.exp(LCs - Lc)
                Snew.append(jnp.exp(LCs) * S0[:, m * DV:(m + 1) * DV]
                            + _dg(Kdec, U, ((0,), (0,))))
            s_ref[h] = jnp.concatenate(Snew, axis=1)
            ohs.append(oh)

        o = jnp.concatenate(ohs, axis=1)
        ms = _f32dot(o * o, blk) * (1.0 / DV)
        o = o * _f32dot(jax.lax.rsqrt(ms + EPS), blkT)
        gate = P[r0:r0 + C, 768:1024]
        out.append(o * (gate * jax.nn.sigmoid(gate)))
    o_ref[...] = jnp.concatenate(out, axis=0)


def _mom_scan(P, wcv):
    T = P.shape[0]
    return pl.pallas_call(
        _scan_body,
        grid=(T // S2,),
        in_specs=[
            pl.BlockSpec((S2, PW), lambda c: (c, 0)),
            pl.BlockSpec((KC, F3), lambda c: (0, 0)),
        ],
        out_specs=pl.BlockSpec((S2, H * DV), lambda c: (c, 0)),
        out_shape=jax.ShapeDtypeStruct((T, H * DV), jnp.float32),
        scratch_shapes=[
            pltpu.VMEM((H, DK, M * DV), jnp.float32),
            pltpu.VMEM((3, F3), jnp.float32),
        ],
    )(P, wcv)


def _mlp_body(og_ref, wo_ref, res_ref, nw_ref, wg_ref, wu_ref, wd_ref,
              out_ref, xn_ref):
    i = pl.program_id(1)

    @pl.when(i == 0)
    def _():
        r = _f32dot(og_ref[...], wo_ref[...]) + res_ref[...]
        xn = r * jax.lax.rsqrt(jnp.mean(r * r, axis=-1, keepdims=True) + EPS)
        xn_ref[...] = xn * nw_ref[...]
        out_ref[...] = r

    xn = xn_ref[...]
    a = _f32dot(xn, wg_ref[...])
    b = _f32dot(xn, wu_ref[...])
    out_ref[...] += _f32dot(a * jax.nn.sigmoid(a) * b, wd_ref[...])


def _mlp(og, wo_s, res, norm_w, mp):
    T = og.shape[0]
    bm = 256
    bi = 512
    return pl.pallas_call(
        _mlp_body,
        grid=(T // bm, I // bi),
        in_specs=[
            pl.BlockSpec((bm, H * DV), lambda t, i: (t, 0)),
            pl.BlockSpec((H * DV, D), lambda t, i: (0, 0)),
            pl.BlockSpec((bm, D), lambda t, i: (t, 0)),
            pl.BlockSpec((1, D), lambda t, i: (0, 0)),
            pl.BlockSpec((D, bi), lambda t, i: (0, i)),
            pl.BlockSpec((D, bi), lambda t, i: (0, i)),
            pl.BlockSpec((bi, D), lambda t, i: (i, 0)),
        ],
        out_specs=pl.BlockSpec((bm, D), lambda t, i: (t, 0)),
        out_shape=jax.ShapeDtypeStruct((T, D), jnp.float32),
        scratch_shapes=[pltpu.VMEM((bm, D), jnp.float32)],
    )(og, wo_s, res, norm_w.reshape(1, D), mp['wg'], mp['wu'], mp['wd'])


def _wcat(ap):
    pad = jnp.zeros((D, PW - (4 * 256 + 3 * M)), jnp.float32)
    return jnp.concatenate(
        [ap['wq'], ap['wk'], ap['wv'], ap['w_gate'],
         ap['w_beta'], ap['w_g'], ap['w_router'], pad], axis=1)


def _wcv(ap):
    return jnp.concatenate(
        [ap['q_conv'], ap['k_conv'], ap['v_conv']], axis=0).T


def _wo_scaled(ap):
    return ap['wo'] * jnp.tile(ap['o_norm_w'], H)[:, None]


def kernel(query, keyval, params):
    p = params
    x0 = query[0]
    kv = keyval[0]
    a1, a2 = p['qa_attn'], p['qkv_attn']

    P1 = _prologue(x0, p['qa_attn_norm'], _wcat(a1))
    OG1 = _mom_scan(P1, _wcv(a1))
    out1 = _mlp(OG1, _wo_scaled(a1), x0, p['qa_mlp_norm'], p['qa_mlp'])

    qkv = jnp.concatenate([kv, out1], axis=0)
    P2 = _prologue(qkv, p['qkv_norm'], _wcat(a2))
    OG2 = _mom_scan(P2, _wcv(a2))
    out2 = _mlp(OG2[kv.shape[0]:], _wo_scaled(a2), out1, p['ffn_norm'],
                p['ffn'])

    q_rl = P1[:, 1032:1032 + M][None]
    kv_rl = P2[:, 1032:1032 + M][None]
    return out2[None], q_rl, kv_rl


# speedup vs baseline: 13.5013x; 1.1964x over previous
import functools

import jax
import jax.numpy as jnp
from jax.experimental import pallas as pl
from jax.experimental.pallas import tpu as pltpu

D = 1024
H = 4
DK = 64
DV = 64
M = 4
KC = 4
I = 2048
EPS = 1e-6
C = 64
MH = M * H
PW = 1152
F3 = 3 * H * DK


def _f32dot(a, b):
    return jnp.dot(a, b, preferred_element_type=jnp.float32)


def _dg(a, b, dims):
    return jax.lax.dot_general(a, b, (dims, ((), ())),
                               preferred_element_type=jnp.float32)


def _prologue_body(x_ref, nw_ref, w_ref, o_ref):
    x = x_ref[...]
    xn = x * jax.lax.rsqrt(jnp.mean(x * x, axis=-1, keepdims=True) + EPS)
    xn = xn * nw_ref[...]
    o_ref[...] = _f32dot(xn, w_ref[...])


def _prologue(x, norm_w, wcat):
    T = x.shape[0]
    bm = 256
    return pl.pallas_call(
        _prologue_body,
        grid=(T // bm,),
        in_specs=[
            pl.BlockSpec((bm, D), lambda i: (i, 0)),
            pl.BlockSpec((1, D), lambda i: (0, 0)),
            pl.BlockSpec((D, PW), lambda i: (0, 0)),
        ],
        out_specs=pl.BlockSpec((bm, PW), lambda i: (i, 0)),
        out_shape=jax.ShapeDtypeStruct((T, PW), jnp.float32),
    )(x, norm_w.reshape(1, D), wcat)


NSUB = 2
S2 = NSUB * C


def _scan_body(p_ref, wcv_ref, o_ref, s0_ref, s1_ref, s2_ref, s3_ref,
               hist_ref):
    c = pl.program_id(0)
    srefs = (s0_ref, s1_ref, s2_ref, s3_ref)

    @pl.when(c == 0)
    def _():
        for sr in srefs:
            sr[...] = jnp.zeros_like(sr)
        hist_ref[...] = jnp.zeros_like(hist_ref)

    P = p_ref[...]
    raw = P[:, 0:F3]
    wcv = wcv_ref[...]

    xp = jnp.concatenate([hist_ref[...], raw], axis=0)
    y = (xp[0:S2] * wcv[0:1] + xp[1:S2 + 1] * wcv[1:2]
         + xp[2:S2 + 2] * wcv[2:3] + xp[3:S2 + 3] * wcv[3:4])
    y = y * jax.nn.sigmoid(y)
    hist_ref[...] = raw[S2 - 3:S2, :]

    qr = y[:, 0:256]
    kr = y[:, 256:512]
    v = y[:, 512:768]

    blk = ((jax.lax.broadcasted_iota(jnp.int32, (256, H), 0) // DK)
           == jax.lax.broadcasted_iota(jnp.int32, (256, H), 1)
           ).astype(jnp.float32)
    blkT = ((jax.lax.broadcasted_iota(jnp.int32, (H, 256), 1) // DK)
            == jax.lax.broadcasted_iota(jnp.int32, (H, 256), 0)
            ).astype(jnp.float32)
    qs = jax.lax.rsqrt(_f32dot(qr * qr, blk) + 1e-6)
    ks = jax.lax.rsqrt(_f32dot(kr * kr, blk) + 1e-6)
    q = qr * _f32dot(qs, blkT)
    k = kr * _f32dot(ks, blkT)

    beta = jax.nn.sigmoid(P[:, 1024:1028])
    g = -jax.nn.softplus(P[:, 1028:1032])
    rl = P[:, 1032:1036]

    m_iota = jax.lax.broadcasted_iota(jnp.int32, (S2, M), 1)
    mx1 = jnp.max(rl, axis=1, keepdims=True)
    idx1 = jnp.min(jnp.where(rl == mx1, m_iota, M), axis=1, keepdims=True)
    one1 = (m_iota == idx1)
    rl2 = jnp.where(one1, -1e30, rl)
    mx2 = jnp.max(rl2, axis=1, keepdims=True)
    idx2 = jnp.min(jnp.where(rl2 == mx2, m_iota, M), axis=1, keepdims=True)
    one2 = (m_iota == idx2)
    e2 = jnp.exp(mx2 - mx1)
    z = 1.0 + e2
    wfull = (one1.astype(jnp.float32) + one2.astype(jnp.float32) * e2) / z
    ind = (one1 | one2).astype(jnp.float32)

    rep = ((jax.lax.broadcasted_iota(jnp.int32, (H, MH), 1) % H)
           == jax.lax.broadcasted_iota(jnp.int32, (H, MH), 0)
           ).astype(jnp.float32)
    mem = ((jax.lax.broadcasted_iota(jnp.int32, (M, MH), 1) // H)
           == jax.lax.broadcasted_iota(jnp.int32, (M, MH), 0)
           ).astype(jnp.float32)
    b16 = _f32dot(beta, rep) * _f32dot(wfull, mem)
    la16 = _f32dot(g, rep) * _f32dot(ind, mem)

    ii2 = jax.lax.broadcasted_iota(jnp.int32, (S2, S2), 0)
    jj2 = jax.lax.broadcasted_iota(jnp.int32, (S2, S2), 1)
    same = (ii2 // C) == (jj2 // C)
    trilb = ((ii2 >= jj2) & same).astype(jnp.float32)
    triub = ((ii2 <= jj2) & same).astype(jnp.float32)
    L16 = _f32dot(trilb, la16)
    LT = _dg(la16, triub, ((0,), (0,)))

    ii = jax.lax.broadcasted_iota(jnp.int32, (C, C), 0)
    jj = jax.lax.broadcasted_iota(jnp.int32, (C, C), 1)
    low = ii >= jj
    lows = ii > jj

    out = []
    for s in range(NSUB):
        r0 = s * C
        ohs = []
        for h in range(H):
            Kh = k[r0:r0 + C, h * DK:(h + 1) * DK]
            Qh = q[r0:r0 + C, h * DK:(h + 1) * DK]
            Vh = v[r0:r0 + C, h * DV:(h + 1) * DV]
            KK = _dg(Kh, Kh, ((1,), (1,)))
            QK = _dg(Qh, Kh, ((1,), (1,)))
            S0 = srefs[h][...]
            QS = _f32dot(Qh, S0)
            oh = jnp.zeros((C, DV), jnp.float32)
            Snew = []
            for m in range(M):
                mh = m * H + h
                Lc = L16[r0:r0 + C, mh:mh + 1]
                Lr = LT[mh:mh + 1, r0:r0 + C]
                bc = b16[r0:r0 + C, mh:mh + 1]
                wm = wfull[r0:r0 + C, m:m + 1]
                eLc = jnp.exp(Lc)
                Dm = jnp.exp(jnp.where(low, Lc - Lr, -1e30))
                N = jnp.where(lows, (-bc) * Dm * KK, 0.0)
                Z = jnp.concatenate([bc * Vh, Kh * (bc * eLc)], axis=1)
                Z = Z + _f32dot(N, Z)
                N2 = _f32dot(N, N)
                Z = Z + _f32dot(N2, Z)
                N4 = _f32dot(N2, N2)
                Z = Z + _f32dot(N4, Z)
                N8 = _f32dot(N4, N4)
                Z = Z + _f32dot(N8, Z)
                N16 = _f32dot(N8, N8)
                Z = Z + _f32dot(N16, Z)
                N32 = _f32dot(N16, N16)
                Z = Z + _f32dot(N32, Z)
                DQK = Dm * QK
                U = Z[:, 0:DV] - _f32dot(Z[:, DV:DV + DK],
                                         S0[:, m * DV:(m + 1) * DV])
                oh = oh + wm * (eLc * QS[:, m * DV:(m + 1) * DV]
                                + _f32dot(DQK, U))
                LCs = L16[r0 + C - 1:r0 + C, mh:mh + 1]
                Kdec = Kh * jnp.exp(LCs - Lc)
                Snew.append(jnp.exp(LCs) * S0[:, m * DV:(m + 1) * DV]
                            + _dg(Kdec, U, ((0,), (0,))))
            srefs[h][...] = jnp.concatenate(Snew, axis=1)
            ohs.append(oh)

        o = jnp.concatenate(ohs, axis=1)
        ms = _f32dot(o * o, blk) * (1.0 / DV)
        o = o * _f32dot(jax.lax.rsqrt(ms + EPS), blkT)
        gate = P[r0:r0 + C, 768:1024]
        out.append(o * (gate * jax.nn.sigmoid(gate)))
    o_ref[...] = jnp.concatenate(out, axis=0)


def _mom_scan(P, wcv):
    T = P.shape[0]
    return pl.pallas_call(
        _scan_body,
        grid=(T // S2,),
        in_specs=[
            pl.BlockSpec((S2, PW), lambda c: (c, 0)),
            pl.BlockSpec((KC, F3), lambda c: (0, 0)),
        ],
        out_specs=pl.BlockSpec((S2, H * DV), lambda c: (c, 0)),
        out_shape=jax.ShapeDtypeStruct((T, H * DV), jnp.float32),
        scratch_shapes=[
            pltpu.VMEM((DK, M * DV), jnp.float32),
            pltpu.VMEM((DK, M * DV), jnp.float32),
            pltpu.VMEM((DK, M * DV), jnp.float32),
            pltpu.VMEM((DK, M * DV), jnp.float32),
            pltpu.VMEM((3, F3), jnp.float32),
        ],
    )(P, wcv)


def _mlp_body(og_ref, wo_ref, res_ref, nw_ref, wg_ref, wu_ref, wd_ref,
              out_ref, xn_ref):
    i = pl.program_id(1)

    @pl.when(i == 0)
    def _():
        r = _f32dot(og_ref[...], wo_ref[...]) + res_ref[...]
        xn = r * jax.lax.rsqrt(jnp.mean(r * r, axis=-1, keepdims=True) + EPS)
        xn_ref[...] = xn * nw_ref[...]
        out_ref[...] = r

    xn = xn_ref[...]
    a = _f32dot(xn, wg_ref[...])
    b = _f32dot(xn, wu_ref[...])
    out_ref[...] += _f32dot(a * jax.nn.sigmoid(a) * b, wd_ref[...])


def _mlp(og, wo_s, res, norm_w, mp):
    T = og.shape[0]
    bm = 256
    bi = 512
    return pl.pallas_call(
        _mlp_body,
        grid=(T // bm, I // bi),
        in_specs=[
            pl.BlockSpec((bm, H * DV), lambda t, i: (t, 0)),
            pl.BlockSpec((H * DV, D), lambda t, i: (0, 0)),
            pl.BlockSpec((bm, D), lambda t, i: (t, 0)),
            pl.BlockSpec((1, D), lambda t, i: (0, 0)),
            pl.BlockSpec((D, bi), lambda t, i: (0, i)),
            pl.BlockSpec((D, bi), lambda t, i: (0, i)),
            pl.BlockSpec((bi, D), lambda t, i: (i, 0)),
        ],
        out_specs=pl.BlockSpec((bm, D), lambda t, i: (t, 0)),
        out_shape=jax.ShapeDtypeStruct((T, D), jnp.float32),
        scratch_shapes=[pltpu.VMEM((bm, D), jnp.float32)],
    )(og, wo_s, res, norm_w.reshape(1, D), mp['wg'], mp['wu'], mp['wd'])


def _wcat(ap):
    pad = jnp.zeros((D, PW - (4 * 256 + 3 * M)), jnp.float32)
    return jnp.concatenate(
        [ap['wq'], ap['wk'], ap['wv'], ap['w_gate'],
         ap['w_beta'], ap['w_g'], ap['w_router'], pad], axis=1)


def _wcv(ap):
    return jnp.concatenate(
        [ap['q_conv'], ap['k_conv'], ap['v_conv']], axis=0).T


def _wo_scaled(ap):
    return ap['wo'] * jnp.tile(ap['o_norm_w'], H)[:, None]


def kernel(query, keyval, params):
    p = params
    x0 = query[0]
    kv = keyval[0]
    a1, a2 = p['qa_attn'], p['qkv_attn']

    P1 = _prologue(x0, p['qa_attn_norm'], _wcat(a1))
    OG1 = _mom_scan(P1, _wcv(a1))
    out1 = _mlp(OG1, _wo_scaled(a1), x0, p['qa_mlp_norm'], p['qa_mlp'])

    qkv = jnp.concatenate([kv, out1], axis=0)
    P2 = _prologue(qkv, p['qkv_norm'], _wcat(a2))
    OG2 = _mom_scan(P2, _wcv(a2))
    out2 = _mlp(OG2[kv.shape[0]:], _wo_scaled(a2), out1, p['ffn_norm'],
                p['ffn'])

    q_rl = P1[:, 1032:1032 + M][None]
    kv_rl = P2[:, 1032:1032 + M][None]
    return out2[None], q_rl, kv_rl


# speedup vs baseline: 19.9093x; 1.4746x over previous
import functools

import jax
import jax.numpy as jnp
from jax.experimental import pallas as pl
from jax.experimental.pallas import tpu as pltpu

D = 1024
H = 4
DK = 64
DV = 64
M = 4
KC = 4
I = 2048
EPS = 1e-6
C = 128
LEV = 7
MH = M * H
PW = 1152
F3 = 3 * H * DK


def _f32dot(a, b):
    return jnp.dot(a, b, preferred_element_type=jnp.float32)


def _dg(a, b, dims):
    return jax.lax.dot_general(a, b, (dims, ((), ())),
                               preferred_element_type=jnp.float32)


def _prologue_body(x_ref, nw_ref, w_ref, o_ref):
    x = x_ref[...]
    xn = x * jax.lax.rsqrt(jnp.mean(x * x, axis=-1, keepdims=True) + EPS)
    xn = xn * nw_ref[...]
    o_ref[...] = _f32dot(xn, w_ref[...])


def _prologue(x, norm_w, wcat):
    T = x.shape[0]
    bm = 256
    return pl.pallas_call(
        _prologue_body,
        grid=(T // bm,),
        in_specs=[
            pl.BlockSpec((bm, D), lambda i: (i, 0)),
            pl.BlockSpec((1, D), lambda i: (0, 0)),
            pl.BlockSpec((D, PW), lambda i: (0, 0)),
        ],
        out_specs=pl.BlockSpec((bm, PW), lambda i: (i, 0)),
        out_shape=jax.ShapeDtypeStruct((T, PW), jnp.float32),
    )(x, norm_w.reshape(1, D), wcat)


NSUB = 2
S2 = NSUB * C


def _scan_body(p_ref, wcv_ref, o_ref, s0_ref, s1_ref, s2_ref, s3_ref,
               hist_ref):
    c = pl.program_id(0)
    srefs = (s0_ref, s1_ref, s2_ref, s3_ref)

    @pl.when(c == 0)
    def _():
        for sr in srefs:
            sr[...] = jnp.zeros_like(sr)
        hist_ref[...] = jnp.zeros_like(hist_ref)

    P = p_ref[...]
    raw = P[:, 0:F3]
    wcv = wcv_ref[...]

    xp = jnp.concatenate([hist_ref[...], raw], axis=0)
    y = (xp[0:S2] * wcv[0:1] + xp[1:S2 + 1] * wcv[1:2]
         + xp[2:S2 + 2] * wcv[2:3] + xp[3:S2 + 3] * wcv[3:4])
    y = y * jax.nn.sigmoid(y)
    hist_ref[...] = raw[S2 - 3:S2, :]

    qr = y[:, 0:256]
    kr = y[:, 256:512]
    v = y[:, 512:768]

    blk = ((jax.lax.broadcasted_iota(jnp.int32, (256, H), 0) // DK)
           == jax.lax.broadcasted_iota(jnp.int32, (256, H), 1)
           ).astype(jnp.float32)
    blkT = ((jax.lax.broadcasted_iota(jnp.int32, (H, 256), 1) // DK)
            == jax.lax.broadcasted_iota(jnp.int32, (H, 256), 0)
            ).astype(jnp.float32)
    qs = jax.lax.rsqrt(_f32dot(qr * qr, blk) + 1e-6)
    ks = jax.lax.rsqrt(_f32dot(kr * kr, blk) + 1e-6)
    q = qr * _f32dot(qs, blkT)
    k = kr * _f32dot(ks, blkT)

    beta = jax.nn.sigmoid(P[:, 1024:1028])
    g = -jax.nn.softplus(P[:, 1028:1032])
    rl = P[:, 1032:1036]

    m_iota = jax.lax.broadcasted_iota(jnp.int32, (S2, M), 1)
    mx1 = jnp.max(rl, axis=1, keepdims=True)
    idx1 = jnp.min(jnp.where(rl == mx1, m_iota, M), axis=1, keepdims=True)
    one1 = (m_iota == idx1)
    rl2 = jnp.where(one1, -1e30, rl)
    mx2 = jnp.max(rl2, axis=1, keepdims=True)
    idx2 = jnp.min(jnp.where(rl2 == mx2, m_iota, M), axis=1, keepdims=True)
    one2 = (m_iota == idx2)
    e2 = jnp.exp(mx2 - mx1)
    z = 1.0 + e2
    wfull = (one1.astype(jnp.float32) + one2.astype(jnp.float32) * e2) / z
    ind = (one1 | one2).astype(jnp.float32)

    rep = ((jax.lax.broadcasted_iota(jnp.int32, (H, MH), 1) % H)
           == jax.lax.broadcasted_iota(jnp.int32, (H, MH), 0)
           ).astype(jnp.float32)
    mem = ((jax.lax.broadcasted_iota(jnp.int32, (M, MH), 1) // H)
           == jax.lax.broadcasted_iota(jnp.int32, (M, MH), 0)
           ).astype(jnp.float32)
    b16 = _f32dot(beta, rep) * _f32dot(wfull, mem)
    la16 = _f32dot(g, rep) * _f32dot(ind, mem)

    ii2 = jax.lax.broadcasted_iota(jnp.int32, (S2, S2), 0)
    jj2 = jax.lax.broadcasted_iota(jnp.int32, (S2, S2), 1)
    same = (ii2 // C) == (jj2 // C)
    trilb = ((ii2 >= jj2) & same).astype(jnp.float32)
    triub = ((ii2 <= jj2) & same).astype(jnp.float32)
    L16 = _f32dot(trilb, la16)
    LT = _dg(la16, triub, ((0,), (0,)))

    ii = jax.lax.broadcasted_iota(jnp.int32, (C, C), 0)
    jj = jax.lax.broadcasted_iota(jnp.int32, (C, C), 1)
    low = ii >= jj
    lows = ii > jj

    out = []
    for s in range(NSUB):
        r0 = s * C
        ohs = []
        for h in range(H):
            Kh = k[r0:r0 + C, h * DK:(h + 1) * DK]
            Qh = q[r0:r0 + C, h * DK:(h + 1) * DK]
            Vh = v[r0:r0 + C, h * DV:(h + 1) * DV]
            KK = _dg(Kh, Kh, ((1,), (1,)))
            QK = _dg(Qh, Kh, ((1,), (1,)))
            S0 = srefs[h][...]
            QS = _f32dot(Qh, S0)
            oh = jnp.zeros((C, DV), jnp.float32)
            Snew = []
            for m in range(M):
                mh = m * H + h
                Lc = L16[r0:r0 + C, mh:mh + 1]
                Lr = LT[mh:mh + 1, r0:r0 + C]
                bc = b16[r0:r0 + C, mh:mh + 1]
                wm = wfull[r0:r0 + C, m:m + 1]
                eLc = jnp.exp(Lc)
                Dm = jnp.exp(jnp.where(low, Lc - Lr, -1e30))
                N = jnp.where(lows, (-bc) * Dm * KK, 0.0)
                Z = jnp.concatenate([bc * Vh, Kh * (bc * eLc)], axis=1)
                Z = Z + _f32dot(N, Z)
                for _ in range(LEV - 1):
                    N = _f32dot(N, N)
                    Z = Z + _f32dot(N, Z)
                DQK = Dm * QK
                U = Z[:, 0:DV] - _f32dot(Z[:, DV:DV + DK],
                                         S0[:, m * DV:(m + 1) * DV])
                oh = oh + wm * (eLc * QS[:, m * DV:(m + 1) * DV]
                                + _f32dot(DQK, U))
                LCs = L16[r0 + C - 1:r0 + C, mh:mh + 1]
                Kdec = Kh * jnp.exp(LCs - Lc)
                Snew.append(jnp.exp(LCs) * S0[:, m * DV:(m + 1) * DV]
                            + _dg(Kdec, U, ((0,), (0,))))
            srefs[h][...] = jnp.concatenate(Snew, axis=1)
            ohs.append(oh)

        o = jnp.concatenate(ohs, axis=1)
        ms = _f32dot(o * o, blk) * (1.0 / DV)
        o = o * _f32dot(jax.lax.rsqrt(ms + EPS), blkT)
        gate = P[r0:r0 + C, 768:1024]
        out.append(o * (gate * jax.nn.sigmoid(gate)))
    o_ref[...] = jnp.concatenate(out, axis=0)


def _mom_scan(P, wcv):
    T = P.shape[0]
    return pl.pallas_call(
        _scan_body,
        grid=(T // S2,),
        in_specs=[
            pl.BlockSpec((S2, PW), lambda c: (c, 0)),
            pl.BlockSpec((KC, F3), lambda c: (0, 0)),
        ],
        out_specs=pl.BlockSpec((S2, H * DV), lambda c: (c, 0)),
        out_shape=jax.ShapeDtypeStruct((T, H * DV), jnp.float32),
        scratch_shapes=[
            pltpu.VMEM((DK, M * DV), jnp.float32),
            pltpu.VMEM((DK, M * DV), jnp.float32),
            pltpu.VMEM((DK, M * DV), jnp.float32),
            pltpu.VMEM((DK, M * DV), jnp.float32),
            pltpu.VMEM((3, F3), jnp.float32),
        ],
    )(P, wcv)


def _mlp_body(og_ref, wo_ref, res_ref, nw_ref, wg_ref, wu_ref, wd_ref,
              out_ref, xn_ref):
    i = pl.program_id(1)

    @pl.when(i == 0)
    def _():
        r = _f32dot(og_ref[...], wo_ref[...]) + res_ref[...]
        xn = r * jax.lax.rsqrt(jnp.mean(r * r, axis=-1, keepdims=True) + EPS)
        xn_ref[...] = xn * nw_ref[...]
        out_ref[...] = r

    xn = xn_ref[...]
    a = _f32dot(xn, wg_ref[...])
    b = _f32dot(xn, wu_ref[...])
    out_ref[...] += _f32dot(a * jax.nn.sigmoid(a) * b, wd_ref[...])


def _mlp(og, wo_s, res, norm_w, mp):
    T = og.shape[0]
    bm = 256
    bi = 512
    return pl.pallas_call(
        _mlp_body,
        grid=(T // bm, I // bi),
        in_specs=[
            pl.BlockSpec((bm, H * DV), lambda t, i: (t, 0)),
            pl.BlockSpec((H * DV, D), lambda t, i: (0, 0)),
            pl.BlockSpec((bm, D), lambda t, i: (t, 0)),
            pl.BlockSpec((1, D), lambda t, i: (0, 0)),
            pl.BlockSpec((D, bi), lambda t, i: (0, i)),
            pl.BlockSpec((D, bi), lambda t, i: (0, i)),
            pl.BlockSpec((bi, D), lambda t, i: (i, 0)),
        ],
        out_specs=pl.BlockSpec((bm, D), lambda t, i: (t, 0)),
        out_shape=jax.ShapeDtypeStruct((T, D), jnp.float32),
        scratch_shapes=[pltpu.VMEM((bm, D), jnp.float32)],
    )(og, wo_s, res, norm_w.reshape(1, D), mp['wg'], mp['wu'], mp['wd'])


def _wcat(ap):
    pad = jnp.zeros((D, PW - (4 * 256 + 3 * M)), jnp.float32)
    return jnp.concatenate(
        [ap['wq'], ap['wk'], ap['wv'], ap['w_gate'],
         ap['w_beta'], ap['w_g'], ap['w_router'], pad], axis=1)


def _wcv(ap):
    return jnp.concatenate(
        [ap['q_conv'], ap['k_conv'], ap['v_conv']], axis=0).T


def _wo_scaled(ap):
    return ap['wo'] * jnp.tile(ap['o_norm_w'], H)[:, None]


def kernel(query, keyval, params):
    p = params
    x0 = query[0]
    kv = keyval[0]
    a1, a2 = p['qa_attn'], p['qkv_attn']

    P1 = _prologue(x0, p['qa_attn_norm'], _wcat(a1))
    OG1 = _mom_scan(P1, _wcv(a1))
    out1 = _mlp(OG1, _wo_scaled(a1), x0, p['qa_mlp_norm'], p['qa_mlp'])

    qkv = jnp.concatenate([kv, out1], axis=0)
    P2 = _prologue(qkv, p['qkv_norm'], _wcat(a2))
    OG2 = _mom_scan(P2, _wcv(a2))
    out2 = _mlp(OG2[kv.shape[0]:], _wo_scaled(a2), out1, p['ffn_norm'],
                p['ffn'])

    q_rl = P1[:, 1032:1032 + M][None]
    kv_rl = P2[:, 1032:1032 + M][None]
    return out2[None], q_rl, kv_rl
